# Initial kernel scaffold; baseline (speedup 1.0000x reference)
import jax
import jax.numpy as jnp
from jax import lax
from jax.experimental import pallas as pl
from jax.experimental.pallas import tpu as pltpu
from jax.experimental.pallas import tpu_sc as plsc

N_ROWS = 320000
N_FEAT = 128
N_SEG = 10000
NC = 2
NS = 16
NW = NC * NS
SUB = 128
TOTAL_SUB = N_ROWS // SUB
BASE_N = TOTAL_SUB // NW
EXTRA = TOTAL_SUB % NW
SEG_PER_TILE = N_SEG // NS


def _sc_body(x_hbm, idx_hbm, zeros_hbm, part_hbm, acc, xbuf, ibuf):
    c = lax.axis_index("c")
    s = lax.axis_index("s")
    wid = c * NS + s

    pltpu.sync_copy(zeros_hbm, acc.at[pl.ds(s * SEG_PER_TILE, SEG_PER_TILE)])
    plsc.subcore_barrier()

    n_sub = BASE_N + jnp.where(wid < EXTRA, 1, 0)
    base_sub = wid * BASE_N + jnp.minimum(wid, EXTRA)

    def body(j, carry):
        sub = base_sub + j
        pltpu.sync_copy(x_hbm.at[pl.ds(sub * SUB, SUB)], xbuf)
        pltpu.sync_copy(idx_hbm.at[sub], ibuf)
        pltpu.sync_copy(xbuf, acc.at[ibuf], add=True)
        return carry

    lax.fori_loop(0, n_sub, body, 0)

    plsc.subcore_barrier()
    pltpu.sync_copy(
        acc.at[pl.ds(s * SEG_PER_TILE, SEG_PER_TILE)],
        part_hbm.at[c, pl.ds(s * SEG_PER_TILE, SEG_PER_TILE)],
    )


def _add_body(a_ref, b_ref, o_ref):
    o_ref[...] = a_ref[...] + b_ref[...]


def kernel(x, idx):
    idx2d = idx.astype(jnp.int32).reshape(TOTAL_SUB, SUB)
    zeros = jnp.zeros((SEG_PER_TILE, N_FEAT), jnp.float32)

    part = pl.kernel(
        _sc_body,
        out_type=jax.ShapeDtypeStruct((NC, N_SEG, N_FEAT), jnp.float32),
        mesh=plsc.VectorSubcoreMesh(core_axis_name="c", subcore_axis_name="s"),
        scratch_types=[
            pltpu.VMEM_SHARED((N_SEG, N_FEAT), jnp.float32),
            pltpu.VMEM((SUB, N_FEAT), jnp.float32),
            pltpu.VMEM((SUB,), jnp.int32),
        ],
    )(x, idx2d, zeros)

    blk = 1000
    out = pl.pallas_call(
        _add_body,
        grid=(N_SEG // blk,),
        in_specs=[
            pl.BlockSpec((blk, N_FEAT), lambda i: (i, 0)),
            pl.BlockSpec((blk, N_FEAT), lambda i: (i, 0)),
        ],
        out_specs=pl.BlockSpec((blk, N_FEAT), lambda i: (i, 0)),
        out_shape=jax.ShapeDtypeStruct((N_SEG, N_FEAT), jnp.float32),
    )(part[0], part[1])
    return out


# speedup vs baseline: 4.4316x; 4.4316x over previous
import jax
import jax.numpy as jnp
from jax import lax
from jax.experimental import pallas as pl
from jax.experimental.pallas import tpu as pltpu
from jax.experimental.pallas import tpu_sc as plsc

N_ROWS = 320000
N_FEAT = 128
N_SEG = 10000
NC = 2
NS = 16
NW = NC * NS
SUB = 128
TOTAL_SUB = N_ROWS // SUB
BASE_N = TOTAL_SUB // NW
EXTRA = TOTAL_SUB % NW
N_SEG_PAD = 10240
SEG_PER_TILE = N_SEG_PAD // NS


def _sc_body(x_hbm, idx_hbm, zeros_hbm, part_hbm, acc, xbuf, ibuf):
    c = lax.axis_index("c")
    s = lax.axis_index("s")
    wid = c * NS + s

    pltpu.sync_copy(zeros_hbm, acc.at[pl.ds(s * SEG_PER_TILE, SEG_PER_TILE)])
    plsc.subcore_barrier()

    n_sub = BASE_N + jnp.where(wid < EXTRA, 1, 0)
    base_sub = wid * BASE_N + jnp.minimum(wid, EXTRA)

    def body(j, carry):
        sub = base_sub + j
        pltpu.sync_copy(x_hbm.at[pl.ds(sub * SUB, SUB)], xbuf)
        pltpu.sync_copy(idx_hbm.at[pl.ds(sub * SUB, SUB)], ibuf)
        pltpu.sync_copy(xbuf, acc.at[ibuf], add=True)
        return carry

    lax.fori_loop(0, n_sub, body, 0)

    plsc.subcore_barrier()
    pltpu.sync_copy(
        acc.at[pl.ds(s * SEG_PER_TILE, SEG_PER_TILE)],
        part_hbm.at[c, pl.ds(s * SEG_PER_TILE, SEG_PER_TILE)],
    )


def _add_body(a_ref, b_ref, o_ref):
    o_ref[...] = a_ref[0] + b_ref[0]


def kernel(x, idx):
    idx1d = idx.astype(jnp.int32)
    zeros = jnp.zeros((SEG_PER_TILE, N_FEAT), jnp.float32)

    part = pl.kernel(
        _sc_body,
        out_type=jax.ShapeDtypeStruct((NC, N_SEG_PAD, N_FEAT), jnp.float32),
        mesh=plsc.VectorSubcoreMesh(core_axis_name="c", subcore_axis_name="s"),
        scratch_types=[
            pltpu.VMEM_SHARED((N_SEG_PAD, N_FEAT), jnp.float32),
            pltpu.VMEM((SUB, N_FEAT), jnp.float32),
            pltpu.VMEM((SUB,), jnp.int32),
        ],
    )(x, idx1d, zeros)

    blk = 1000
    out = pl.pallas_call(
        _add_body,
        grid=(N_SEG // blk,),
        in_specs=[
            pl.BlockSpec((1, blk, N_FEAT), lambda i: (0, i, 0)),
            pl.BlockSpec((1, blk, N_FEAT), lambda i: (1, i, 0)),
        ],
        out_specs=pl.BlockSpec((blk, N_FEAT), lambda i: (i, 0)),
        out_shape=jax.ShapeDtypeStruct((N_SEG, N_FEAT), jnp.float32),
    )(part, part)
    return out


# speedup vs baseline: 7.4866x; 1.6894x over previous
import jax
import jax.numpy as jnp
from jax import lax
from jax.experimental import pallas as pl
from jax.experimental.pallas import tpu as pltpu
from jax.experimental.pallas import tpu_sc as plsc

N_ROWS = 320000
N_FEAT = 128
N_SEG = 10000
NC = 2
NS = 16
NW = NC * NS
SUB = 128
TOTAL_SUB = N_ROWS // SUB
BASE_N = TOTAL_SUB // NW
EXTRA = TOTAL_SUB % NW
N_SEG_PAD = 10240
SEG_PER_TILE = N_SEG_PAD // NS


def _sc_body(x_hbm, idx_hbm, zeros_hbm, part_hbm, acc, xbuf, ibuf, sems):
    c = lax.axis_index("c")
    s = lax.axis_index("s")
    wid = c * NS + s

    pltpu.sync_copy(zeros_hbm, acc.at[pl.ds(s * SEG_PER_TILE, SEG_PER_TILE)])
    plsc.subcore_barrier()

    base_sub = wid * BASE_N + jnp.minimum(wid, EXTRA)

    def start_load(g, slot):
        sub = base_sub + g
        pltpu.async_copy(
            x_hbm.at[pl.ds(sub * SUB, SUB)], xbuf.at[slot], sems.at[slot])
        pltpu.async_copy(
            idx_hbm.at[pl.ds(sub * SUB, SUB)], ibuf.at[slot], sems.at[slot])

    def wait_load(slot):
        pltpu.make_async_copy(
            x_hbm.at[pl.ds(0, SUB)], xbuf.at[slot], sems.at[slot]).wait()
        pltpu.make_async_copy(
            idx_hbm.at[pl.ds(0, SUB)], ibuf.at[slot], sems.at[slot]).wait()

    start_load(0, 0)
    start_load(1, 1)

    def outer(gbase, carry):
        for b in range(2):
            g = gbase + b
            wait_load(b)
            pltpu.sync_copy(xbuf.at[b], acc.at[ibuf.at[b]], add=True)

            @pl.when(g + 2 < BASE_N)
            def _():
                start_load(g + 2, b)

        return carry

    lax.fori_loop(0, BASE_N // 2, lambda i, cr: outer(i * 2, cr), 0)

    @pl.when(wid < EXTRA)
    def _():
        sub = base_sub + BASE_N
        pltpu.sync_copy(x_hbm.at[pl.ds(sub * SUB, SUB)], xbuf.at[0])
        pltpu.sync_copy(idx_hbm.at[pl.ds(sub * SUB, SUB)], ibuf.at[0])
        pltpu.sync_copy(xbuf.at[0], acc.at[ibuf.at[0]], add=True)

    plsc.subcore_barrier()
    pltpu.sync_copy(
        acc.at[pl.ds(s * SEG_PER_TILE, SEG_PER_TILE)],
        part_hbm.at[c, pl.ds(s * SEG_PER_TILE, SEG_PER_TILE)],
    )


def _add_body(a_ref, b_ref, o_ref):
    o_ref[...] = a_ref[0] + b_ref[0]


def kernel(x, idx):
    idx1d = idx.astype(jnp.int32)
    zeros = jnp.zeros((SEG_PER_TILE, N_FEAT), jnp.float32)

    part = pl.kernel(
        _sc_body,
        out_type=jax.ShapeDtypeStruct((NC, N_SEG_PAD, N_FEAT), jnp.float32),
        mesh=plsc.VectorSubcoreMesh(core_axis_name="c", subcore_axis_name="s"),
        scratch_types=[
            pltpu.VMEM_SHARED((N_SEG_PAD, N_FEAT), jnp.float32),
            pltpu.VMEM((2, SUB, N_FEAT), jnp.float32),
            pltpu.VMEM((2, SUB), jnp.int32),
            pltpu.SemaphoreType.DMA((2,)),
        ],
    )(x, idx1d, zeros)

    blk = 1000
    out = pl.pallas_call(
        _add_body,
        grid=(N_SEG // blk,),
        in_specs=[
            pl.BlockSpec((1, blk, N_FEAT), lambda i: (0, i, 0)),
            pl.BlockSpec((1, blk, N_FEAT), lambda i: (1, i, 0)),
        ],
        out_specs=pl.BlockSpec((blk, N_FEAT), lambda i: (i, 0)),
        out_shape=jax.ShapeDtypeStruct((N_SEG, N_FEAT), jnp.float32),
    )(part, part)
    return out
